# Initial kernel scaffold; baseline (speedup 1.0000x reference)
import jax
import jax.numpy as jnp
from jax.experimental import pallas as pl


def kernel(x, pos, edge_index, batch, W1, a1s, a1d, b1, W2, a2s, a2d, b2, W3, a3s, a3d, b3, Wl, bl):
    raise NotImplementedError("write your pallas kernel here")


# speedup vs baseline: 48.0733x; 48.0733x over previous
import jax
import jax.numpy as jnp
from jax import lax
from jax.experimental import pallas as pl
from jax.experimental.pallas import tpu as pltpu
from jax.experimental.pallas import tpu_sc as plsc

N = 10000
NPAD = 10240
D = 128
H = 8
C = 8
HID = 64
HH = 32
TW = 48
NCLS = 10
NBATCH = 64
E = 320000
ETOT = E + N
CH = 512
NB = CH // 128
CHUNKS = 42
EPAD = 16 * CHUNKS * CH
EROWS = EPAD // 128
RPT = NPAD // 16
RB = 1000
GRID = N // RB

_f32 = jnp.float32
_i32 = jnp.int32

_mesh = plsc.VectorSubcoreMesh(
    core_axis_name="c", subcore_axis_name="s", num_cores=2, num_subcores=16)


def _edge_body(t0_hbm, t1_hbm, a_hbm, src_hbm, dst_hbm,
               o0_hbm, o1_hbm,
               idx_s, idx_d, arow, drow, hrow, obuf, acc_sh,
               sem_a, sem_h):
    cid = lax.axis_index("c")
    sid = lax.axis_index("s")
    iota = lax.broadcasted_iota(_i32, (16,), 0)
    swap = iota ^ 8
    hsel = iota >> 3
    hsel0 = hsel + cid * 4
    hsel1 = hsel0 + 2
    zero = jnp.zeros((16,), _f32)

    def zloop(r, carry):
        for g in range(TW // 16):
            obuf[r, pl.ds(g * 16, 16)] = zero
        return carry
    lax.fori_loop(0, RPT, zloop, 0)
    pltpu.sync_copy(obuf, acc_sh.at[pl.ds(sid * RPT, RPT)])
    plsc.subcore_barrier()

    def chunk(k, carry):
        rowbase = (sid * CHUNKS + k) * NB
        ebase = rowbase * 128
        pltpu.sync_copy(src_hbm.at[pl.ds(rowbase, NB)], idx_s)
        pltpu.sync_copy(dst_hbm.at[pl.ds(rowbase, NB)], idx_d)
        cps = [pltpu.async_copy(a_hbm.at[idx_s.at[j]],
                                arow.at[pl.ds(j * 128, 128)], sem_a)
               for j in range(NB)]
        cps += [pltpu.async_copy(a_hbm.at[idx_d.at[j]],
                                 drow.at[pl.ds(j * 128, 128)], sem_a)
                for j in range(NB)]

        @pl.when(cid == 0)
        def _():
            for j in range(NB):
                pltpu.async_copy(t0_hbm.at[idx_s.at[j]],
                                 hrow.at[pl.ds(j * 128, 128)], sem_h)

        @pl.when(cid == 1)
        def _():
            for j in range(NB):
                pltpu.async_copy(t1_hbm.at[idx_s.at[j]],
                                 hrow.at[pl.ds(j * 128, 128)], sem_h)

        for cp in cps:
            cp.wait()
        drain = pltpu.make_async_copy(t0_hbm.at[pl.ds(0, 128)],
                                      hrow.at[pl.ds(0, 128)], sem_h)
        for _ in range(NB):
            drain.wait()

        def lanes(e, c2):
            ad = jnp.take_along_axis(drow[e], swap, axis=0)
            s = arow[e] + ad
            lr = jnp.maximum(s, 0.0) + 0.2 * jnp.minimum(s, 0.0)
            validf = jnp.where(ebase + e < ETOT, 1.0, 0.0)
            ex = jnp.exp(lr) * validf
            w0 = jnp.take_along_axis(ex, hsel0, axis=0)
            w1 = jnp.take_along_axis(ex, hsel1, axis=0)
            hrow[e, pl.ds(0, 16)] = hrow[e, pl.ds(0, 16)] * w0
            hrow[e, pl.ds(16, 16)] = hrow[e, pl.ds(16, 16)] * w1
            hrow[e, pl.ds(32, 16)] = hrow[e, pl.ds(32, 16)] * ex
            return c2
        lax.fori_loop(0, CH, lanes, 0)

        for j in range(NB):
            pltpu.sync_copy(hrow.at[pl.ds(j * 128, 128)],
                            acc_sh.at[idx_d.at[j]], add=True)
        return carry
    lax.fori_loop(0, CHUNKS, chunk, 0)
    plsc.subcore_barrier()

    pltpu.sync_copy(acc_sh.at[pl.ds(sid * RPT, RPT)], obuf)

    @pl.when(cid == 0)
    def _():
        pltpu.sync_copy(obuf, o0_hbm.at[pl.ds(sid * RPT, RPT)])

    @pl.when(cid == 1)
    def _():
        pltpu.sync_copy(obuf, o1_hbm.at[pl.ds(sid * RPT, RPT)])


_edge_layer = pl.kernel(
    _edge_body,
    out_type=(
        jax.ShapeDtypeStruct((NPAD, TW), _f32),
        jax.ShapeDtypeStruct((NPAD, TW), _f32),
    ),
    mesh=_mesh,
    compiler_params=pltpu.CompilerParams(use_tc_tiling_on_sc=False),
    scratch_types=[
        pltpu.VMEM((NB, 128), _i32),
        pltpu.VMEM((NB, 128), _i32),
        pltpu.VMEM((CH, 16), _f32),
        pltpu.VMEM((CH, 16), _f32),
        pltpu.VMEM((CH, TW), _f32),
        pltpu.VMEM((RPT, TW), _f32),
        pltpu.VMEM_SHARED((NPAD, TW), _f32),
        pltpu.SemaphoreType.DMA,
        pltpu.SemaphoreType.DMA,
    ],
)


def _emit_tables(h, t0_ref, t1_ref, a_ref, m_ref):
    ones8 = jnp.ones((h.shape[0], 8), _f32)
    zero8 = jnp.zeros((h.shape[0], 8), _f32)
    t0_ref[...] = jnp.concatenate([h[:, :HH], ones8, zero8], axis=1)
    t1_ref[...] = jnp.concatenate([h[:, HH:], ones8, zero8], axis=1)
    a_ref[...] = jnp.dot(h, m_ref[...], preferred_element_type=_f32)


def _tc1_body(x_ref, w_ref, m_ref, t0_ref, t1_ref, a_ref):
    h = jnp.dot(x_ref[...], w_ref[...], preferred_element_type=_f32)
    _emit_tables(h, t0_ref, t1_ref, a_ref, m_ref)


def _tc1(x, W, M):
    return pl.pallas_call(
        _tc1_body,
        grid=(GRID,),
        in_specs=[pl.BlockSpec((RB, D), lambda i: (i, 0)),
                  pl.BlockSpec((D, HID), lambda i: (0, 0)),
                  pl.BlockSpec((HID, 2 * H), lambda i: (0, 0))],
        out_specs=[pl.BlockSpec((RB, TW), lambda i: (i, 0)),
                   pl.BlockSpec((RB, TW), lambda i: (i, 0)),
                   pl.BlockSpec((RB, 2 * H), lambda i: (i, 0))],
        out_shape=[jax.ShapeDtypeStruct((N, TW), _f32),
                   jax.ShapeDtypeStruct((N, TW), _f32),
                   jax.ShapeDtypeStruct((N, 2 * H), _f32)],
    )(x, W, M)


def _normalize(o0_ref, o1_ref, x8_ref, b_ref):
    num = jnp.concatenate([o0_ref[...][:, :HH], o1_ref[...][:, :HH]], axis=1)
    dn = o0_ref[...][:, HH:HH + H] + 1e-16
    dn64 = jnp.dot(dn, x8_ref[...], preferred_element_type=_f32)
    v = num / dn64 + b_ref[...]
    return jnp.where(v > 0.0, v, jnp.exp(v) - 1.0)


def _tc23_body(o0_ref, o1_ref, x8_ref, b_ref, w_ref, m_ref,
               t0_ref, t1_ref, a_ref):
    y = _normalize(o0_ref, o1_ref, x8_ref, b_ref)
    h = jnp.dot(y, w_ref[...], preferred_element_type=_f32)
    _emit_tables(h, t0_ref, t1_ref, a_ref, m_ref)


def _tc23(o0, o1, X8, b, W, M):
    return pl.pallas_call(
        _tc23_body,
        grid=(GRID,),
        in_specs=[pl.BlockSpec((RB, TW), lambda i: (i, 0)),
                  pl.BlockSpec((RB, TW), lambda i: (i, 0)),
                  pl.BlockSpec((H, HID), lambda i: (0, 0)),
                  pl.BlockSpec((1, HID), lambda i: (0, 0)),
                  pl.BlockSpec((HID, HID), lambda i: (0, 0)),
                  pl.BlockSpec((HID, 2 * H), lambda i: (0, 0))],
        out_specs=[pl.BlockSpec((RB, TW), lambda i: (i, 0)),
                   pl.BlockSpec((RB, TW), lambda i: (i, 0)),
                   pl.BlockSpec((RB, 2 * H), lambda i: (i, 0))],
        out_shape=[jax.ShapeDtypeStruct((N, TW), _f32),
                   jax.ShapeDtypeStruct((N, TW), _f32),
                   jax.ShapeDtypeStruct((N, 2 * H), _f32)],
    )(o0, o1, X8, b, W, M)


def _pool_body(o0_ref, o1_ref, x8_ref, b_ref, bt_ref, wl_ref, bl_ref,
               out_ref, pooled, cnt):
    i = pl.program_id(0)

    @pl.when(i == 0)
    def _():
        pooled[...] = jnp.zeros_like(pooled)
        cnt[...] = jnp.zeros_like(cnt)

    y = _normalize(o0_ref, o1_ref, x8_ref, b_ref)
    bt = bt_ref[0]
    oh = (lax.broadcasted_iota(_i32, (NBATCH, RB), 0)
          == jnp.broadcast_to(bt, (NBATCH, RB))).astype(_f32)
    pooled[...] += jnp.dot(oh, y, preferred_element_type=_f32)
    cnt[...] += jnp.dot(oh, jnp.ones((RB, 128), _f32),
                        preferred_element_type=_f32)

    @pl.when(i == GRID - 1)
    def _():
        g = pooled[...] / jnp.maximum(cnt[...][:, 0:1], 1.0)
        out_ref[...] = (jnp.dot(g, wl_ref[...], preferred_element_type=_f32)
                        + bl_ref[...])


def _pool(o0, o1, X8, b, batch_r, Wl, bl):
    return pl.pallas_call(
        _pool_body,
        grid=(GRID,),
        in_specs=[pl.BlockSpec((RB, TW), lambda i: (i, 0)),
                  pl.BlockSpec((RB, TW), lambda i: (i, 0)),
                  pl.BlockSpec((H, HID), lambda i: (0, 0)),
                  pl.BlockSpec((1, HID), lambda i: (0, 0)),
                  pl.BlockSpec((1, 1, RB), lambda i: (i, 0, 0)),
                  pl.BlockSpec((HID, NCLS), lambda i: (0, 0)),
                  pl.BlockSpec((1, NCLS), lambda i: (0, 0))],
        out_specs=pl.BlockSpec((NBATCH, NCLS), lambda i: (0, 0)),
        out_shape=jax.ShapeDtypeStruct((NBATCH, NCLS), _f32),
        scratch_shapes=[pltpu.VMEM((NBATCH, HID), _f32),
                        pltpu.VMEM((NBATCH, 128), _f32)],
    )(o0, o1, X8, b, batch_r, Wl, bl)


def _amat(a_s, a_d):
    r = jnp.arange(HID)
    M = jnp.zeros((HID, 2 * H), _f32)
    M = M.at[r, r // C].set(a_s.reshape(HID))
    M = M.at[r, H + r // C].set(a_d.reshape(HID))
    return M


def kernel(x, pos, edge_index, batch, W1, a1s, a1d, b1, W2, a2s, a2d, b2,
           W3, a3s, a3d, b3, Wl, bl):
    loop = jnp.arange(N, dtype=_i32)
    padz = jnp.zeros((EPAD - ETOT,), _i32)
    src = jnp.concatenate([edge_index[0].astype(_i32), loop, padz])
    dst = jnp.concatenate([edge_index[1].astype(_i32), loop, padz])
    src = src.reshape(EROWS, 128)
    dst = dst.reshape(EROWS, 128)
    r = jnp.arange(HID)
    X8 = (jnp.zeros((H, HID), _f32).at[r // C, r].set(1.0))

    t0, t1, A = _tc1(x, W1, _amat(a1s, a1d))
    o0, o1 = _edge_layer(t0, t1, A, src, dst)
    t0, t1, A = _tc23(o0, o1, X8, b1.reshape(1, HID), W2, _amat(a2s, a2d))
    o0, o1 = _edge_layer(t0, t1, A, src, dst)
    t0, t1, A = _tc23(o0, o1, X8, b2.reshape(1, HID), W3, _amat(a3s, a3d))
    o0, o1 = _edge_layer(t0, t1, A, src, dst)
    batch_r = batch.astype(_i32).reshape(GRID, 1, RB)
    return _pool(o0, o1, X8, b3.reshape(1, HID), batch_r, Wl,
                 bl.reshape(1, NCLS))


# speedup vs baseline: 76.5894x; 1.5932x over previous
import jax
import jax.numpy as jnp
from jax import lax
from jax.experimental import pallas as pl
from jax.experimental.pallas import tpu as pltpu
from jax.experimental.pallas import tpu_sc as plsc

N = 10000
NPAD = 10240
D = 128
H = 8
C = 8
HID = 64
HH = 32
TW = 48
NCLS = 10
NBATCH = 64
E = 320000
ETOT = E + N
CH = 512
NB = CH // 128
CHUNKS = 42
EPAD = 16 * CHUNKS * CH
EROWS = EPAD // 128
RPT = NPAD // 16
RB = 1000
GRID = N // RB

_f32 = jnp.float32
_i32 = jnp.int32

_mesh = plsc.VectorSubcoreMesh(
    core_axis_name="c", subcore_axis_name="s", num_cores=2, num_subcores=16)


def _edge_body(t0_hbm, t1_hbm, a_hbm, src_hbm, dst_hbm,
               o0_hbm, o1_hbm,
               idx_s, idx_d, arow, drow, hrow, obuf, acc_sh,
               sem_a, sem_h):
    cid = lax.axis_index("c")
    sid = lax.axis_index("s")
    iota = lax.broadcasted_iota(_i32, (16,), 0)
    swap = iota ^ 8
    hsel = iota >> 3
    hsel0 = hsel + cid * 4
    hsel1 = hsel0 + 2
    zero = jnp.zeros((16,), _f32)

    @plsc.parallel_loop(0, RPT, unroll=4)
    def _(r):
        for g in range(TW // 16):
            obuf[r, pl.ds(g * 16, 16)] = zero
    pltpu.sync_copy(obuf, acc_sh.at[pl.ds(sid * RPT, RPT)])
    plsc.subcore_barrier()

    def chunk(k, carry):
        rowbase = (sid * CHUNKS + k) * NB
        ebase = rowbase * 128
        pltpu.sync_copy(src_hbm.at[pl.ds(rowbase, NB)], idx_s)
        pltpu.sync_copy(dst_hbm.at[pl.ds(rowbase, NB)], idx_d)
        cps = [pltpu.async_copy(a_hbm.at[idx_s.at[j]],
                                arow.at[pl.ds(j * 128, 128)], sem_a)
               for j in range(NB)]
        cps += [pltpu.async_copy(a_hbm.at[idx_d.at[j]],
                                 drow.at[pl.ds(j * 128, 128)], sem_a)
                for j in range(NB)]

        @pl.when(cid == 0)
        def _():
            for j in range(NB):
                pltpu.async_copy(t0_hbm.at[idx_s.at[j]],
                                 hrow.at[pl.ds(j * 128, 128)], sem_h)

        @pl.when(cid == 1)
        def _():
            for j in range(NB):
                pltpu.async_copy(t1_hbm.at[idx_s.at[j]],
                                 hrow.at[pl.ds(j * 128, 128)], sem_h)

        for cp in cps:
            cp.wait()
        drain = pltpu.make_async_copy(t0_hbm.at[pl.ds(0, 128)],
                                      hrow.at[pl.ds(0, 128)], sem_h)
        for _ in range(NB):
            drain.wait()

        @plsc.parallel_loop(0, CH, unroll=8)
        def _(e):
            ad = jnp.take_along_axis(drow[e], swap, axis=0)
            s = arow[e] + ad
            lr = jnp.maximum(s, 0.0) + 0.2 * jnp.minimum(s, 0.0)
            validf = jnp.where(ebase + e < ETOT, 1.0, 0.0)
            ex = jnp.exp(lr) * validf
            w0 = jnp.take_along_axis(ex, hsel0, axis=0)
            w1 = jnp.take_along_axis(ex, hsel1, axis=0)
            hrow[e, pl.ds(0, 16)] = hrow[e, pl.ds(0, 16)] * w0
            hrow[e, pl.ds(16, 16)] = hrow[e, pl.ds(16, 16)] * w1
            hrow[e, pl.ds(32, 16)] = hrow[e, pl.ds(32, 16)] * ex

        for j in range(NB):
            pltpu.sync_copy(hrow.at[pl.ds(j * 128, 128)],
                            acc_sh.at[idx_d.at[j]], add=True)
        return carry
    lax.fori_loop(0, CHUNKS, chunk, 0)
    plsc.subcore_barrier()

    pltpu.sync_copy(acc_sh.at[pl.ds(sid * RPT, RPT)], obuf)

    @pl.when(cid == 0)
    def _():
        pltpu.sync_copy(obuf, o0_hbm.at[pl.ds(sid * RPT, RPT)])

    @pl.when(cid == 1)
    def _():
        pltpu.sync_copy(obuf, o1_hbm.at[pl.ds(sid * RPT, RPT)])


_edge_layer = pl.kernel(
    _edge_body,
    out_type=(
        jax.ShapeDtypeStruct((NPAD, TW), _f32),
        jax.ShapeDtypeStruct((NPAD, TW), _f32),
    ),
    mesh=_mesh,
    compiler_params=pltpu.CompilerParams(use_tc_tiling_on_sc=False),
    scratch_types=[
        pltpu.VMEM((NB, 128), _i32),
        pltpu.VMEM((NB, 128), _i32),
        pltpu.VMEM((CH, 16), _f32),
        pltpu.VMEM((CH, 16), _f32),
        pltpu.VMEM((CH, TW), _f32),
        pltpu.VMEM((RPT, TW), _f32),
        pltpu.VMEM_SHARED((NPAD, TW), _f32),
        pltpu.SemaphoreType.DMA,
        pltpu.SemaphoreType.DMA,
    ],
)


def _emit_tables(h, t0_ref, t1_ref, a_ref, m_ref):
    ones8 = jnp.ones((h.shape[0], 8), _f32)
    zero8 = jnp.zeros((h.shape[0], 8), _f32)
    t0_ref[...] = jnp.concatenate([h[:, :HH], ones8, zero8], axis=1)
    t1_ref[...] = jnp.concatenate([h[:, HH:], ones8, zero8], axis=1)
    a_ref[...] = jnp.dot(h, m_ref[...], preferred_element_type=_f32)


def _tc1_body(x_ref, w_ref, m_ref, t0_ref, t1_ref, a_ref):
    h = jnp.dot(x_ref[...], w_ref[...], preferred_element_type=_f32)
    _emit_tables(h, t0_ref, t1_ref, a_ref, m_ref)


def _tc1(x, W, M):
    return pl.pallas_call(
        _tc1_body,
        grid=(GRID,),
        in_specs=[pl.BlockSpec((RB, D), lambda i: (i, 0)),
                  pl.BlockSpec((D, HID), lambda i: (0, 0)),
                  pl.BlockSpec((HID, 2 * H), lambda i: (0, 0))],
        out_specs=[pl.BlockSpec((RB, TW), lambda i: (i, 0)),
                   pl.BlockSpec((RB, TW), lambda i: (i, 0)),
                   pl.BlockSpec((RB, 2 * H), lambda i: (i, 0))],
        out_shape=[jax.ShapeDtypeStruct((N, TW), _f32),
                   jax.ShapeDtypeStruct((N, TW), _f32),
                   jax.ShapeDtypeStruct((N, 2 * H), _f32)],
    )(x, W, M)


def _normalize(o0_ref, o1_ref, x8_ref, b_ref):
    num = jnp.concatenate([o0_ref[...][:, :HH], o1_ref[...][:, :HH]], axis=1)
    dn = o0_ref[...][:, HH:HH + H] + 1e-16
    dn64 = jnp.dot(dn, x8_ref[...], preferred_element_type=_f32)
    v = num / dn64 + b_ref[...]
    return jnp.where(v > 0.0, v, jnp.exp(v) - 1.0)


def _tc23_body(o0_ref, o1_ref, x8_ref, b_ref, w_ref, m_ref,
               t0_ref, t1_ref, a_ref):
    y = _normalize(o0_ref, o1_ref, x8_ref, b_ref)
    h = jnp.dot(y, w_ref[...], preferred_element_type=_f32)
    _emit_tables(h, t0_ref, t1_ref, a_ref, m_ref)


def _tc23(o0, o1, X8, b, W, M):
    return pl.pallas_call(
        _tc23_body,
        grid=(GRID,),
        in_specs=[pl.BlockSpec((RB, TW), lambda i: (i, 0)),
                  pl.BlockSpec((RB, TW), lambda i: (i, 0)),
                  pl.BlockSpec((H, HID), lambda i: (0, 0)),
                  pl.BlockSpec((1, HID), lambda i: (0, 0)),
                  pl.BlockSpec((HID, HID), lambda i: (0, 0)),
                  pl.BlockSpec((HID, 2 * H), lambda i: (0, 0))],
        out_specs=[pl.BlockSpec((RB, TW), lambda i: (i, 0)),
                   pl.BlockSpec((RB, TW), lambda i: (i, 0)),
                   pl.BlockSpec((RB, 2 * H), lambda i: (i, 0))],
        out_shape=[jax.ShapeDtypeStruct((N, TW), _f32),
                   jax.ShapeDtypeStruct((N, TW), _f32),
                   jax.ShapeDtypeStruct((N, 2 * H), _f32)],
    )(o0, o1, X8, b, W, M)


def _pool_body(o0_ref, o1_ref, x8_ref, b_ref, bt_ref, wl_ref, bl_ref,
               out_ref, pooled, cnt):
    i = pl.program_id(0)

    @pl.when(i == 0)
    def _():
        pooled[...] = jnp.zeros_like(pooled)
        cnt[...] = jnp.zeros_like(cnt)

    y = _normalize(o0_ref, o1_ref, x8_ref, b_ref)
    bt = bt_ref[0]
    oh = (lax.broadcasted_iota(_i32, (NBATCH, RB), 0)
          == jnp.broadcast_to(bt, (NBATCH, RB))).astype(_f32)
    pooled[...] += jnp.dot(oh, y, preferred_element_type=_f32)
    cnt[...] += jnp.dot(oh, jnp.ones((RB, 128), _f32),
                        preferred_element_type=_f32)

    @pl.when(i == GRID - 1)
    def _():
        g = pooled[...] / jnp.maximum(cnt[...][:, 0:1], 1.0)
        out_ref[...] = (jnp.dot(g, wl_ref[...], preferred_element_type=_f32)
                        + bl_ref[...])


def _pool(o0, o1, X8, b, batch_r, Wl, bl):
    return pl.pallas_call(
        _pool_body,
        grid=(GRID,),
        in_specs=[pl.BlockSpec((RB, TW), lambda i: (i, 0)),
                  pl.BlockSpec((RB, TW), lambda i: (i, 0)),
                  pl.BlockSpec((H, HID), lambda i: (0, 0)),
                  pl.BlockSpec((1, HID), lambda i: (0, 0)),
                  pl.BlockSpec((1, 1, RB), lambda i: (i, 0, 0)),
                  pl.BlockSpec((HID, NCLS), lambda i: (0, 0)),
                  pl.BlockSpec((1, NCLS), lambda i: (0, 0))],
        out_specs=pl.BlockSpec((NBATCH, NCLS), lambda i: (0, 0)),
        out_shape=jax.ShapeDtypeStruct((NBATCH, NCLS), _f32),
        scratch_shapes=[pltpu.VMEM((NBATCH, HID), _f32),
                        pltpu.VMEM((NBATCH, 128), _f32)],
    )(o0, o1, X8, b, batch_r, Wl, bl)


def _amat(a_s, a_d):
    r = jnp.arange(HID)
    M = jnp.zeros((HID, 2 * H), _f32)
    M = M.at[r, r // C].set(a_s.reshape(HID))
    M = M.at[r, H + r // C].set(a_d.reshape(HID))
    return M


def kernel(x, pos, edge_index, batch, W1, a1s, a1d, b1, W2, a2s, a2d, b2,
           W3, a3s, a3d, b3, Wl, bl):
    loop = jnp.arange(N, dtype=_i32)
    padz = jnp.zeros((EPAD - ETOT,), _i32)
    src = jnp.concatenate([edge_index[0].astype(_i32), loop, padz])
    dst = jnp.concatenate([edge_index[1].astype(_i32), loop, padz])
    src = src.reshape(EROWS, 128)
    dst = dst.reshape(EROWS, 128)
    r = jnp.arange(HID)
    X8 = (jnp.zeros((H, HID), _f32).at[r // C, r].set(1.0))

    t0, t1, A = _tc1(x, W1, _amat(a1s, a1d))
    o0, o1 = _edge_layer(t0, t1, A, src, dst)
    t0, t1, A = _tc23(o0, o1, X8, b1.reshape(1, HID), W2, _amat(a2s, a2d))
    o0, o1 = _edge_layer(t0, t1, A, src, dst)
    t0, t1, A = _tc23(o0, o1, X8, b2.reshape(1, HID), W3, _amat(a3s, a3d))
    o0, o1 = _edge_layer(t0, t1, A, src, dst)
    batch_r = batch.astype(_i32).reshape(GRID, 1, RB)
    return _pool(o0, o1, X8, b3.reshape(1, HID), batch_r, Wl,
                 bl.reshape(1, NCLS))


# speedup vs baseline: 92.9181x; 1.2132x over previous
import jax
import jax.numpy as jnp
from jax import lax
from jax.experimental import pallas as pl
from jax.experimental.pallas import tpu as pltpu
from jax.experimental.pallas import tpu_sc as plsc

N = 10000
NPAD = 10240
D = 128
H = 8
C = 8
HID = 64
HH = 32
TW = 48
NCLS = 10
NBATCH = 64
E = 320000
ETOT = E + N
CH = 512
NB = CH // 128
CHUNKS = 42
EPAD = 16 * CHUNKS * CH
EROWS = EPAD // 128
RPT = NPAD // 16
RB = 1000
GRID = N // RB

_f32 = jnp.float32
_i32 = jnp.int32

_mesh = plsc.VectorSubcoreMesh(
    core_axis_name="c", subcore_axis_name="s", num_cores=2, num_subcores=16)


def _edge_body(t0_hbm, t1_hbm, a_hbm, src_hbm, dst_hbm,
               o0_hbm, o1_hbm,
               idx_s0, idx_d0, arow0, drow0, hrow0,
               idx_s1, idx_d1, arow1, drow1, hrow1,
               acc_sh, sem_a0, sem_h0, sem_a1, sem_h1):
    cid = lax.axis_index("c")
    sid = lax.axis_index("s")
    iota = lax.broadcasted_iota(_i32, (16,), 0)
    swap = iota ^ 8
    hsel = iota >> 3
    hsel0 = hsel + cid * 4
    hsel1 = hsel0 + 2
    zero = jnp.zeros((16,), _f32)
    bufs = ((idx_s0, idx_d0, arow0, drow0, hrow0, sem_a0, sem_h0),
            (idx_s1, idx_d1, arow1, drow1, hrow1, sem_a1, sem_h1))

    @plsc.parallel_loop(0, CH, unroll=4)
    def _(r):
        for g in range(TW // 16):
            hrow0[r, pl.ds(g * 16, 16)] = zero
    pltpu.sync_copy(hrow0, acc_sh.at[pl.ds(sid * RPT, CH)])
    pltpu.sync_copy(hrow0.at[pl.ds(0, RPT - CH)],
                    acc_sh.at[pl.ds(sid * RPT + CH, RPT - CH)])
    plsc.subcore_barrier()

    def fire(k, b):
        idx_s, idx_d, arow, drow, hrow, sem_a, sem_h = bufs[b]
        rowbase = (sid * CHUNKS + k) * NB
        pltpu.sync_copy(src_hbm.at[pl.ds(rowbase, NB)], idx_s)
        pltpu.sync_copy(dst_hbm.at[pl.ds(rowbase, NB)], idx_d)
        for j in range(NB):
            pltpu.async_copy(a_hbm.at[idx_s.at[j]],
                             arow.at[pl.ds(j * 128, 128)], sem_a)
            pltpu.async_copy(a_hbm.at[idx_d.at[j]],
                             drow.at[pl.ds(j * 128, 128)], sem_a)

        @pl.when(cid == 0)
        def _():
            for j in range(NB):
                pltpu.async_copy(t0_hbm.at[idx_s.at[j]],
                                 hrow.at[pl.ds(j * 128, 128)], sem_h)

        @pl.when(cid == 1)
        def _():
            for j in range(NB):
                pltpu.async_copy(t1_hbm.at[idx_s.at[j]],
                                 hrow.at[pl.ds(j * 128, 128)], sem_h)

    def process(k, b):
        idx_s, idx_d, arow, drow, hrow, sem_a, sem_h = bufs[b]
        da = pltpu.make_async_copy(a_hbm.at[pl.ds(0, 128)],
                                   arow.at[pl.ds(0, 128)], sem_a)
        for _ in range(2 * NB):
            da.wait()
        dh = pltpu.make_async_copy(t0_hbm.at[pl.ds(0, 128)],
                                   hrow.at[pl.ds(0, 128)], sem_h)
        for _ in range(NB):
            dh.wait()
        ebase = (sid * CHUNKS + k) * CH

        @plsc.parallel_loop(0, CH, unroll=8)
        def _(e):
            ad = jnp.take_along_axis(drow[e], swap, axis=0)
            s = arow[e] + ad
            lr = jnp.maximum(s, 0.0) + 0.2 * jnp.minimum(s, 0.0)
            validf = jnp.where(ebase + e < ETOT, 1.0, 0.0)
            ex = jnp.exp(lr) * validf
            w0 = jnp.take_along_axis(ex, hsel0, axis=0)
            w1 = jnp.take_along_axis(ex, hsel1, axis=0)
            hrow[e, pl.ds(0, 16)] = hrow[e, pl.ds(0, 16)] * w0
            hrow[e, pl.ds(16, 16)] = hrow[e, pl.ds(16, 16)] * w1
            hrow[e, pl.ds(32, 16)] = hrow[e, pl.ds(32, 16)] * ex

        for j in range(NB):
            pltpu.sync_copy(hrow.at[pl.ds(j * 128, 128)],
                            acc_sh.at[idx_d.at[j]], add=True)

    fire(0, 0)

    def body(kk, carry):
        fire(2 * kk + 1, 1)
        process(2 * kk, 0)

        @pl.when(kk < CHUNKS // 2 - 1)
        def _():
            fire(2 * kk + 2, 0)
        process(2 * kk + 1, 1)
        return carry
    lax.fori_loop(0, CHUNKS // 2, body, 0)
    plsc.subcore_barrier()

    pltpu.sync_copy(acc_sh.at[pl.ds(sid * RPT, CH)], hrow0)
    pltpu.sync_copy(acc_sh.at[pl.ds(sid * RPT + CH, RPT - CH)],
                    hrow1.at[pl.ds(0, RPT - CH)])

    @pl.when(cid == 0)
    def _():
        pltpu.sync_copy(hrow0, o0_hbm.at[pl.ds(sid * RPT, CH)])
        pltpu.sync_copy(hrow1.at[pl.ds(0, RPT - CH)],
                        o0_hbm.at[pl.ds(sid * RPT + CH, RPT - CH)])

    @pl.when(cid == 1)
    def _():
        pltpu.sync_copy(hrow0, o1_hbm.at[pl.ds(sid * RPT, CH)])
        pltpu.sync_copy(hrow1.at[pl.ds(0, RPT - CH)],
                        o1_hbm.at[pl.ds(sid * RPT + CH, RPT - CH)])


_edge_layer = pl.kernel(
    _edge_body,
    out_type=(
        jax.ShapeDtypeStruct((NPAD, TW), _f32),
        jax.ShapeDtypeStruct((NPAD, TW), _f32),
    ),
    mesh=_mesh,
    compiler_params=pltpu.CompilerParams(use_tc_tiling_on_sc=False),
    scratch_types=[
        pltpu.VMEM((NB, 128), _i32),
        pltpu.VMEM((NB, 128), _i32),
        pltpu.VMEM((CH, 16), _f32),
        pltpu.VMEM((CH, 16), _f32),
        pltpu.VMEM((CH, TW), _f32),
        pltpu.VMEM((NB, 128), _i32),
        pltpu.VMEM((NB, 128), _i32),
        pltpu.VMEM((CH, 16), _f32),
        pltpu.VMEM((CH, 16), _f32),
        pltpu.VMEM((CH, TW), _f32),
        pltpu.VMEM_SHARED((NPAD, TW), _f32),
        pltpu.SemaphoreType.DMA,
        pltpu.SemaphoreType.DMA,
        pltpu.SemaphoreType.DMA,
        pltpu.SemaphoreType.DMA,
    ],
)


def _emit_tables(h, t0_ref, t1_ref, a_ref, m_ref):
    ones8 = jnp.ones((h.shape[0], 8), _f32)
    zero8 = jnp.zeros((h.shape[0], 8), _f32)
    t0_ref[...] = jnp.concatenate([h[:, :HH], ones8, zero8], axis=1)
    t1_ref[...] = jnp.concatenate([h[:, HH:], ones8, zero8], axis=1)
    a_ref[...] = jnp.dot(h, m_ref[...], preferred_element_type=_f32)


def _tc1_body(x_ref, w_ref, m_ref, t0_ref, t1_ref, a_ref):
    h = jnp.dot(x_ref[...], w_ref[...], preferred_element_type=_f32)
    _emit_tables(h, t0_ref, t1_ref, a_ref, m_ref)


def _tc1(x, W, M):
    return pl.pallas_call(
        _tc1_body,
        grid=(GRID,),
        in_specs=[pl.BlockSpec((RB, D), lambda i: (i, 0)),
                  pl.BlockSpec((D, HID), lambda i: (0, 0)),
                  pl.BlockSpec((HID, 2 * H), lambda i: (0, 0))],
        out_specs=[pl.BlockSpec((RB, TW), lambda i: (i, 0)),
                   pl.BlockSpec((RB, TW), lambda i: (i, 0)),
                   pl.BlockSpec((RB, 2 * H), lambda i: (i, 0))],
        out_shape=[jax.ShapeDtypeStruct((N, TW), _f32),
                   jax.ShapeDtypeStruct((N, TW), _f32),
                   jax.ShapeDtypeStruct((N, 2 * H), _f32)],
    )(x, W, M)


def _normalize(o0_ref, o1_ref, x8_ref, b_ref):
    num = jnp.concatenate([o0_ref[...][:, :HH], o1_ref[...][:, :HH]], axis=1)
    dn = o0_ref[...][:, HH:HH + H] + 1e-16
    dn64 = jnp.dot(dn, x8_ref[...], preferred_element_type=_f32)
    v = num / dn64 + b_ref[...]
    return jnp.where(v > 0.0, v, jnp.exp(v) - 1.0)


def _tc23_body(o0_ref, o1_ref, x8_ref, b_ref, w_ref, m_ref,
               t0_ref, t1_ref, a_ref):
    y = _normalize(o0_ref, o1_ref, x8_ref, b_ref)
    h = jnp.dot(y, w_ref[...], preferred_element_type=_f32)
    _emit_tables(h, t0_ref, t1_ref, a_ref, m_ref)


def _tc23(o0, o1, X8, b, W, M):
    return pl.pallas_call(
        _tc23_body,
        grid=(GRID,),
        in_specs=[pl.BlockSpec((RB, TW), lambda i: (i, 0)),
                  pl.BlockSpec((RB, TW), lambda i: (i, 0)),
                  pl.BlockSpec((H, HID), lambda i: (0, 0)),
                  pl.BlockSpec((1, HID), lambda i: (0, 0)),
                  pl.BlockSpec((HID, HID), lambda i: (0, 0)),
                  pl.BlockSpec((HID, 2 * H), lambda i: (0, 0))],
        out_specs=[pl.BlockSpec((RB, TW), lambda i: (i, 0)),
                   pl.BlockSpec((RB, TW), lambda i: (i, 0)),
                   pl.BlockSpec((RB, 2 * H), lambda i: (i, 0))],
        out_shape=[jax.ShapeDtypeStruct((N, TW), _f32),
                   jax.ShapeDtypeStruct((N, TW), _f32),
                   jax.ShapeDtypeStruct((N, 2 * H), _f32)],
    )(o0, o1, X8, b, W, M)


def _pool_body(o0_ref, o1_ref, x8_ref, b_ref, bt_ref, wl_ref, bl_ref,
               out_ref, pooled, cnt):
    i = pl.program_id(0)

    @pl.when(i == 0)
    def _():
        pooled[...] = jnp.zeros_like(pooled)
        cnt[...] = jnp.zeros_like(cnt)

    y = _normalize(o0_ref, o1_ref, x8_ref, b_ref)
    bt = bt_ref[0]
    oh = (lax.broadcasted_iota(_i32, (NBATCH, RB), 0)
          == jnp.broadcast_to(bt, (NBATCH, RB))).astype(_f32)
    pooled[...] += jnp.dot(oh, y, preferred_element_type=_f32)
    cnt[...] += jnp.dot(oh, jnp.ones((RB, 128), _f32),
                        preferred_element_type=_f32)

    @pl.when(i == GRID - 1)
    def _():
        g = pooled[...] / jnp.maximum(cnt[...][:, 0:1], 1.0)
        out_ref[...] = (jnp.dot(g, wl_ref[...], preferred_element_type=_f32)
                        + bl_ref[...])


def _pool(o0, o1, X8, b, batch_r, Wl, bl):
    return pl.pallas_call(
        _pool_body,
        grid=(GRID,),
        in_specs=[pl.BlockSpec((RB, TW), lambda i: (i, 0)),
                  pl.BlockSpec((RB, TW), lambda i: (i, 0)),
                  pl.BlockSpec((H, HID), lambda i: (0, 0)),
                  pl.BlockSpec((1, HID), lambda i: (0, 0)),
                  pl.BlockSpec((1, 1, RB), lambda i: (i, 0, 0)),
                  pl.BlockSpec((HID, NCLS), lambda i: (0, 0)),
                  pl.BlockSpec((1, NCLS), lambda i: (0, 0))],
        out_specs=pl.BlockSpec((NBATCH, NCLS), lambda i: (0, 0)),
        out_shape=jax.ShapeDtypeStruct((NBATCH, NCLS), _f32),
        scratch_shapes=[pltpu.VMEM((NBATCH, HID), _f32),
                        pltpu.VMEM((NBATCH, 128), _f32)],
    )(o0, o1, X8, b, batch_r, Wl, bl)


def _amat(a_s, a_d):
    r = jnp.arange(HID)
    M = jnp.zeros((HID, 2 * H), _f32)
    M = M.at[r, r // C].set(a_s.reshape(HID))
    M = M.at[r, H + r // C].set(a_d.reshape(HID))
    return M


def kernel(x, pos, edge_index, batch, W1, a1s, a1d, b1, W2, a2s, a2d, b2,
           W3, a3s, a3d, b3, Wl, bl):
    loop = jnp.arange(N, dtype=_i32)
    padz = jnp.zeros((EPAD - ETOT,), _i32)
    src = jnp.concatenate([edge_index[0].astype(_i32), loop, padz])
    dst = jnp.concatenate([edge_index[1].astype(_i32), loop, padz])
    src = src.reshape(EROWS, 128)
    dst = dst.reshape(EROWS, 128)
    r = jnp.arange(HID)
    X8 = (jnp.zeros((H, HID), _f32).at[r // C, r].set(1.0))

    t0, t1, A = _tc1(x, W1, _amat(a1s, a1d))
    o0, o1 = _edge_layer(t0, t1, A, src, dst)
    t0, t1, A = _tc23(o0, o1, X8, b1.reshape(1, HID), W2, _amat(a2s, a2d))
    o0, o1 = _edge_layer(t0, t1, A, src, dst)
    t0, t1, A = _tc23(o0, o1, X8, b2.reshape(1, HID), W3, _amat(a3s, a3d))
    o0, o1 = _edge_layer(t0, t1, A, src, dst)
    batch_r = batch.astype(_i32).reshape(GRID, 1, RB)
    return _pool(o0, o1, X8, b3.reshape(1, HID), batch_r, Wl,
                 bl.reshape(1, NCLS))


# speedup vs baseline: 97.2875x; 1.0470x over previous
import jax
import jax.numpy as jnp
from jax import lax
from jax.experimental import pallas as pl
from jax.experimental.pallas import tpu as pltpu
from jax.experimental.pallas import tpu_sc as plsc

N = 10000
NPAD = 10240
D = 128
H = 8
C = 8
HID = 64
HH = 32
TW = 48
NCLS = 10
NBATCH = 64
E = 320000
ETOT = E + N
CH = 512
NB = CH // 128
CHUNKS = 42
EPAD = 16 * CHUNKS * CH
EROWS = EPAD // 128
RPT = NPAD // 16
RB = 2000
GRID = N // RB

_f32 = jnp.float32
_i32 = jnp.int32

_mesh = plsc.VectorSubcoreMesh(
    core_axis_name="c", subcore_axis_name="s", num_cores=2, num_subcores=16)


def _edge_body(t0_hbm, t1_hbm, a_hbm, a2_hbm, src_hbm, dst_hbm,
               o0_hbm, o1_hbm,
               idx_s0, idx_d0, arow0, drow0, hrow0,
               idx_s1, idx_d1, arow1, drow1, hrow1,
               acc_sh, sem_a0, sem_h0, sem_w0, sem_a1, sem_h1, sem_w1):
    cid = lax.axis_index("c")
    sid = lax.axis_index("s")
    iota = lax.broadcasted_iota(_i32, (16,), 0)
    hsel = iota >> 3
    hsel0 = hsel + cid * 4
    hsel1 = hsel0 + 2
    zero = jnp.zeros((16,), _f32)
    bufs = ((idx_s0, idx_d0, arow0, drow0, hrow0, sem_a0, sem_h0, sem_w0),
            (idx_s1, idx_d1, arow1, drow1, hrow1, sem_a1, sem_h1, sem_w1))

    @plsc.parallel_loop(0, CH, unroll=4)
    def _(r):
        for g in range(TW // 16):
            hrow0[r, pl.ds(g * 16, 16)] = zero
    pltpu.sync_copy(hrow0, acc_sh.at[pl.ds(sid * RPT, CH)])
    pltpu.sync_copy(hrow0.at[pl.ds(0, RPT - CH)],
                    acc_sh.at[pl.ds(sid * RPT + CH, RPT - CH)])
    plsc.subcore_barrier()

    def fire(k, b, drain_scatter):
        idx_s, idx_d, arow, drow, hrow, sem_a, sem_h, sem_w = bufs[b]
        if drain_scatter:
            dw = pltpu.make_async_copy(hrow.at[pl.ds(0, 128)],
                                       acc_sh.at[pl.ds(0, 128)], sem_w)
            for _ in range(NB):
                dw.wait()
        rowbase = (sid * CHUNKS + k) * NB
        pltpu.sync_copy(src_hbm.at[pl.ds(rowbase, NB)], idx_s)
        pltpu.sync_copy(dst_hbm.at[pl.ds(rowbase, NB)], idx_d)
        for j in range(NB):
            pltpu.async_copy(a_hbm.at[idx_s.at[j]],
                             arow.at[pl.ds(j * 128, 128)], sem_a)
            pltpu.async_copy(a2_hbm.at[idx_d.at[j]],
                             drow.at[pl.ds(j * 128, 128)], sem_a)

        @pl.when(cid == 0)
        def _():
            for j in range(NB):
                pltpu.async_copy(t0_hbm.at[idx_s.at[j]],
                                 hrow.at[pl.ds(j * 128, 128)], sem_h)

        @pl.when(cid == 1)
        def _():
            for j in range(NB):
                pltpu.async_copy(t1_hbm.at[idx_s.at[j]],
                                 hrow.at[pl.ds(j * 128, 128)], sem_h)

    def process(b):
        idx_s, idx_d, arow, drow, hrow, sem_a, sem_h, sem_w = bufs[b]
        da = pltpu.make_async_copy(a_hbm.at[pl.ds(0, 128)],
                                   arow.at[pl.ds(0, 128)], sem_a)
        for _ in range(2 * NB):
            da.wait()
        dh = pltpu.make_async_copy(t0_hbm.at[pl.ds(0, 128)],
                                   hrow.at[pl.ds(0, 128)], sem_h)
        for _ in range(NB):
            dh.wait()

        @plsc.parallel_loop(0, CH, unroll=16)
        def _(e):
            s = arow[e] + drow[e]
            lr = jnp.maximum(s, 0.2 * s)
            ex = jnp.exp(lr)
            w0 = jnp.take_along_axis(ex, hsel0, axis=0)
            w1 = jnp.take_along_axis(ex, hsel1, axis=0)
            hrow[e, pl.ds(0, 16)] = hrow[e, pl.ds(0, 16)] * w0
            hrow[e, pl.ds(16, 16)] = hrow[e, pl.ds(16, 16)] * w1
            hrow[e, pl.ds(32, 16)] = hrow[e, pl.ds(32, 16)] * ex

        for j in range(NB):
            pltpu.async_copy(hrow.at[pl.ds(j * 128, 128)],
                             acc_sh.at[idx_d.at[j]], sem_w, add=True)

    fire(0, 0, False)
    fire(1, 1, False)

    def body(kk, carry):
        process(0)

        @pl.when(kk < CHUNKS // 2 - 1)
        def _():
            fire(2 * kk + 2, 0, True)
        process(1)

        @pl.when(kk < CHUNKS // 2 - 1)
        def _():
            fire(2 * kk + 3, 1, True)
        return carry
    lax.fori_loop(0, CHUNKS // 2, body, 0)
    for b in (0, 1):
        dw = pltpu.make_async_copy(bufs[b][4].at[pl.ds(0, 128)],
                                   acc_sh.at[pl.ds(0, 128)], bufs[b][7])
        for _ in range(NB):
            dw.wait()
    plsc.subcore_barrier()

    pltpu.sync_copy(acc_sh.at[pl.ds(sid * RPT, CH)], hrow0)
    pltpu.sync_copy(acc_sh.at[pl.ds(sid * RPT + CH, RPT - CH)],
                    hrow1.at[pl.ds(0, RPT - CH)])

    @pl.when(cid == 0)
    def _():
        pltpu.sync_copy(hrow0, o0_hbm.at[pl.ds(sid * RPT, CH)])
        pltpu.sync_copy(hrow1.at[pl.ds(0, RPT - CH)],
                        o0_hbm.at[pl.ds(sid * RPT + CH, RPT - CH)])

    @pl.when(cid == 1)
    def _():
        pltpu.sync_copy(hrow0, o1_hbm.at[pl.ds(sid * RPT, CH)])
        pltpu.sync_copy(hrow1.at[pl.ds(0, RPT - CH)],
                        o1_hbm.at[pl.ds(sid * RPT + CH, RPT - CH)])


_edge_layer = pl.kernel(
    _edge_body,
    out_type=(
        jax.ShapeDtypeStruct((NPAD, TW), _f32),
        jax.ShapeDtypeStruct((NPAD, TW), _f32),
    ),
    mesh=_mesh,
    compiler_params=pltpu.CompilerParams(use_tc_tiling_on_sc=False),
    scratch_types=[
        pltpu.VMEM((NB, 128), _i32),
        pltpu.VMEM((NB, 128), _i32),
        pltpu.VMEM((CH, 16), _f32),
        pltpu.VMEM((CH, 16), _f32),
        pltpu.VMEM((CH, TW), _f32),
        pltpu.VMEM((NB, 128), _i32),
        pltpu.VMEM((NB, 128), _i32),
        pltpu.VMEM((CH, 16), _f32),
        pltpu.VMEM((CH, 16), _f32),
        pltpu.VMEM((CH, TW), _f32),
        pltpu.VMEM_SHARED((NPAD, TW), _f32),
        pltpu.SemaphoreType.DMA,
        pltpu.SemaphoreType.DMA,
        pltpu.SemaphoreType.DMA,
        pltpu.SemaphoreType.DMA,
        pltpu.SemaphoreType.DMA,
        pltpu.SemaphoreType.DMA,
    ],
)


def _emit_tables(h, t0_ref, t1_ref, a_ref, a2_ref, m_ref):
    ones8 = jnp.ones((h.shape[0], 8), _f32)
    zero8 = jnp.zeros((h.shape[0], 8), _f32)
    t0_ref[...] = jnp.concatenate([h[:, :HH], ones8, zero8], axis=1)
    t1_ref[...] = jnp.concatenate([h[:, HH:], ones8, zero8], axis=1)
    a = jnp.dot(h, m_ref[...], preferred_element_type=_f32)
    a_ref[...] = a
    a2_ref[...] = jnp.concatenate([a[:, H:], a[:, :H]], axis=1)


def _tc1_body(x_ref, w_ref, m_ref, t0_ref, t1_ref, a_ref, a2_ref):
    h = jnp.dot(x_ref[...], w_ref[...], preferred_element_type=_f32)
    _emit_tables(h, t0_ref, t1_ref, a_ref, a2_ref, m_ref)


def _tc1(x, W, M):
    return pl.pallas_call(
        _tc1_body,
        grid=(GRID,),
        in_specs=[pl.BlockSpec((RB, D), lambda i: (i, 0)),
                  pl.BlockSpec((D, HID), lambda i: (0, 0)),
                  pl.BlockSpec((HID, 2 * H), lambda i: (0, 0))],
        out_specs=[pl.BlockSpec((RB, TW), lambda i: (i, 0)),
                   pl.BlockSpec((RB, TW), lambda i: (i, 0)),
                   pl.BlockSpec((RB, 2 * H), lambda i: (i, 0)),
                   pl.BlockSpec((RB, 2 * H), lambda i: (i, 0))],
        out_shape=[jax.ShapeDtypeStruct((N, TW), _f32),
                   jax.ShapeDtypeStruct((N, TW), _f32),
                   jax.ShapeDtypeStruct((N, 2 * H), _f32),
                   jax.ShapeDtypeStruct((N, 2 * H), _f32)],
    )(x, W, M)


def _normalize(o0_ref, o1_ref, x8_ref, b_ref):
    num = jnp.concatenate([o0_ref[...][:, :HH], o1_ref[...][:, :HH]], axis=1)
    dn = o0_ref[...][:, HH:HH + H] + 1e-16
    dn64 = jnp.dot(dn, x8_ref[...], preferred_element_type=_f32)
    v = num / dn64 + b_ref[...]
    return jnp.where(v > 0.0, v, jnp.exp(v) - 1.0)


def _tc23_body(o0_ref, o1_ref, x8_ref, b_ref, w_ref, m_ref,
               t0_ref, t1_ref, a_ref, a2_ref):
    y = _normalize(o0_ref, o1_ref, x8_ref, b_ref)
    h = jnp.dot(y, w_ref[...], preferred_element_type=_f32)
    _emit_tables(h, t0_ref, t1_ref, a_ref, a2_ref, m_ref)


def _tc23(o0, o1, X8, b, W, M):
    return pl.pallas_call(
        _tc23_body,
        grid=(GRID,),
        in_specs=[pl.BlockSpec((RB, TW), lambda i: (i, 0)),
                  pl.BlockSpec((RB, TW), lambda i: (i, 0)),
                  pl.BlockSpec((H, HID), lambda i: (0, 0)),
                  pl.BlockSpec((1, HID), lambda i: (0, 0)),
                  pl.BlockSpec((HID, HID), lambda i: (0, 0)),
                  pl.BlockSpec((HID, 2 * H), lambda i: (0, 0))],
        out_specs=[pl.BlockSpec((RB, TW), lambda i: (i, 0)),
                   pl.BlockSpec((RB, TW), lambda i: (i, 0)),
                   pl.BlockSpec((RB, 2 * H), lambda i: (i, 0)),
                   pl.BlockSpec((RB, 2 * H), lambda i: (i, 0))],
        out_shape=[jax.ShapeDtypeStruct((N, TW), _f32),
                   jax.ShapeDtypeStruct((N, TW), _f32),
                   jax.ShapeDtypeStruct((N, 2 * H), _f32),
                   jax.ShapeDtypeStruct((N, 2 * H), _f32)],
    )(o0, o1, X8, b, W, M)


def _pool_body(o0_ref, o1_ref, x8_ref, b_ref, bt_ref, wl_ref, bl_ref,
               out_ref, pooled, cnt):
    i = pl.program_id(0)

    @pl.when(i == 0)
    def _():
        pooled[...] = jnp.zeros_like(pooled)
        cnt[...] = jnp.zeros_like(cnt)

    y = _normalize(o0_ref, o1_ref, x8_ref, b_ref)
    bt = bt_ref[0]
    oh = (lax.broadcasted_iota(_i32, (NBATCH, RB), 0)
          == jnp.broadcast_to(bt, (NBATCH, RB))).astype(_f32)
    pooled[...] += jnp.dot(oh, y, preferred_element_type=_f32)
    cnt[...] += jnp.dot(oh, jnp.ones((RB, 128), _f32),
                        preferred_element_type=_f32)

    @pl.when(i == GRID - 1)
    def _():
        g = pooled[...] / jnp.maximum(cnt[...][:, 0:1], 1.0)
        out_ref[...] = (jnp.dot(g, wl_ref[...], preferred_element_type=_f32)
                        + bl_ref[...])


def _pool(o0, o1, X8, b, batch_r, Wl, bl):
    return pl.pallas_call(
        _pool_body,
        grid=(GRID,),
        in_specs=[pl.BlockSpec((RB, TW), lambda i: (i, 0)),
                  pl.BlockSpec((RB, TW), lambda i: (i, 0)),
                  pl.BlockSpec((H, HID), lambda i: (0, 0)),
                  pl.BlockSpec((1, HID), lambda i: (0, 0)),
                  pl.BlockSpec((1, 1, RB), lambda i: (i, 0, 0)),
                  pl.BlockSpec((HID, NCLS), lambda i: (0, 0)),
                  pl.BlockSpec((1, NCLS), lambda i: (0, 0))],
        out_specs=pl.BlockSpec((NBATCH, NCLS), lambda i: (0, 0)),
        out_shape=jax.ShapeDtypeStruct((NBATCH, NCLS), _f32),
        scratch_shapes=[pltpu.VMEM((NBATCH, HID), _f32),
                        pltpu.VMEM((NBATCH, 128), _f32)],
    )(o0, o1, X8, b, batch_r, Wl, bl)


def _amat(a_s, a_d):
    r = jnp.arange(HID)
    M = jnp.zeros((HID, 2 * H), _f32)
    M = M.at[r, r // C].set(a_s.reshape(HID))
    M = M.at[r, H + r // C].set(a_d.reshape(HID))
    return M


def kernel(x, pos, edge_index, batch, W1, a1s, a1d, b1, W2, a2s, a2d, b2,
           W3, a3s, a3d, b3, Wl, bl):
    loop = jnp.arange(N, dtype=_i32)
    padz = jnp.zeros((EPAD - ETOT,), _i32)
    padd = jnp.full((EPAD - ETOT,), NPAD - 1, _i32)
    src = jnp.concatenate([edge_index[0].astype(_i32), loop, padz])
    dst = jnp.concatenate([edge_index[1].astype(_i32), loop, padd])
    src = src.reshape(EROWS, 128)
    dst = dst.reshape(EROWS, 128)
    r = jnp.arange(HID)
    X8 = (jnp.zeros((H, HID), _f32).at[r // C, r].set(1.0))

    t0, t1, A, A2 = _tc1(x, W1, _amat(a1s, a1d))
    o0, o1 = _edge_layer(t0, t1, A, A2, src, dst)
    t0, t1, A, A2 = _tc23(o0, o1, X8, b1.reshape(1, HID), W2,
                          _amat(a2s, a2d))
    o0, o1 = _edge_layer(t0, t1, A, A2, src, dst)
    t0, t1, A, A2 = _tc23(o0, o1, X8, b2.reshape(1, HID), W3,
                          _amat(a3s, a3d))
    o0, o1 = _edge_layer(t0, t1, A, A2, src, dst)
    batch_r = batch.astype(_i32).reshape(GRID, 1, RB)
    return _pool(o0, o1, X8, b3.reshape(1, HID), batch_r, Wl,
                 bl.reshape(1, NCLS))
